# Initial kernel scaffold; baseline (speedup 1.0000x reference)
import functools

import jax
import jax.numpy as jnp
from jax import lax
from jax.experimental import pallas as pl
from jax.experimental.pallas import tpu as pltpu
from jax.experimental.pallas import tpu_sc as plsc

B, L = 16384, 50
VOCAB, D = 10000, 32
LP = 56
NW = 32
BPW = B // NW
NBUF = 8
HALF = D // 2


def _sc_kernel(tok_hbm, table_hbm, out_hbm, tok_v, out_v, t0_v, bufs, sems):
    wid = lax.axis_index("s") * 2 + lax.axis_index("c")
    base = wid * BPW

    pltpu.sync_copy(tok_hbm.at[pl.ds(base, BPW)], tok_v)
    pltpu.sync_copy(table_hbm.at[pl.ds(0, 1)], t0_v)
    t0a = t0_v[0, pl.ds(0, HALF)]
    t0b = t0_v[0, pl.ds(HALF, HALF)]
    lanes = lax.iota(jnp.int32, HALF)

    def fire(r, b):
        pltpu.async_copy(table_hbm.at[tok_v.at[r]], bufs[b], sems[b])

    for b in range(NBUF):
        fire(b, b)

    def step(g, carry):
        for b in range(NBUF):
            r = g * NBUF + b
            buf, sem = bufs[b], sems[b]
            pltpu.make_async_copy(table_hbm.at[tok_v.at[r]], buf, sem).wait()

            pa = [buf[j, pl.ds(0, HALF)] for j in range(4)]
            pb = [buf[j, pl.ds(HALF, HALF)] for j in range(4)]
            for j in range(4, LP):
                pa[j % 4] = pa[j % 4] + buf[j, pl.ds(0, HALF)]
                pb[j % 4] = pb[j % 4] + buf[j, pl.ds(HALF, HALF)]
            acc_a = (pa[0] + pa[1]) + (pa[2] + pa[3])
            acc_b = (pb[0] + pb[1]) + (pb[2] + pb[3])

            @pl.when(r + NBUF < BPW)
            def _():
                fire(r + NBUF, b)

            v0 = tok_v[r, pl.ds(0, 16)]
            v1 = tok_v[r, pl.ds(16, 16)]
            v2 = tok_v[r, pl.ds(32, 16)]
            v3 = tok_v[r, pl.ds(40, 16)]
            ones = jnp.ones((16,), jnp.float32)
            zero = jnp.zeros((16,), jnp.float32)
            c = jnp.where(v0 != 0, ones, zero)
            c = c + jnp.where(v1 != 0, ones, zero)
            c = c + jnp.where(v2 != 0, ones, zero)
            c = c + jnp.where((v3 != 0) & (lanes >= 8), ones, zero)
            cnt = jnp.sum(c)

            n0 = jnp.float32(LP) - cnt
            scale = 1.0 / jnp.maximum(cnt, 1.0)
            out_v[r, pl.ds(0, HALF)] = (acc_a - n0 * t0a) * scale
            out_v[r, pl.ds(HALF, HALF)] = (acc_b - n0 * t0b) * scale
        return carry

    lax.fori_loop(0, BPW // NBUF, step, 0)
    pltpu.sync_copy(out_v, out_hbm.at[pl.ds(base, BPW)])


@jax.jit
def kernel(tokens, table):
    tok_pad = jnp.pad(tokens, ((0, 0), (0, LP - L)))
    mesh = plsc.VectorSubcoreMesh(core_axis_name="c", subcore_axis_name="s")
    f = pl.kernel(
        _sc_kernel,
        out_type=jax.ShapeDtypeStruct((B, D), jnp.float32),
        mesh=mesh,
        scratch_types=[
            pltpu.VMEM((BPW, LP), jnp.int32),
            pltpu.VMEM((BPW, D), jnp.float32),
            pltpu.VMEM((1, D), jnp.float32),
            [pltpu.VMEM((LP, D), jnp.float32) for _ in range(NBUF)],
            [pltpu.SemaphoreType.DMA for _ in range(NBUF)],
        ],
    )
    return f(tok_pad, table)


# speedup vs baseline: 19.3920x; 19.3920x over previous
import jax
import jax.numpy as jnp
from jax import lax
from jax.experimental import pallas as pl
from jax.experimental.pallas import tpu as pltpu
from jax.experimental.pallas import tpu_sc as plsc

B, L = 16384, 50
VOCAB, D = 10000, 32
C = 8
NCG = D // C
NBS = 32 // NCG
RPT = B // NBS
CHUNK = 256
NCHUNK = RPT // CHUNK
TSL = VOCAB * C
TCH = L * CHUNK
OCH = CHUNK * C


def _sc_body(tok_hbm, tab_hbm, out_hbm, ts, tb, ob, tsem, osem):
    wid = lax.axis_index("s") * 2 + lax.axis_index("c")
    cg = lax.rem(wid, NCG)
    bs = lax.div(wid, NCG)
    lanes = lax.iota(jnp.int32, 16)

    pltpu.sync_copy(tab_hbm.at[pl.ds(cg * TSL, TSL)], ts)
    head = ts[pl.ds(0, 16)]
    ts[pl.ds(0, 16)] = jnp.where(lanes < C, 0.0, head)

    def fire_tok(k, par):
        pltpu.async_copy(
            tok_hbm.at[pl.ds((bs * NCHUNK + k) * TCH, TCH)], tb[par], tsem[par])

    def out_slice(k):
        return out_hbm.at[pl.ds((cg * NBS * NCHUNK + bs * NCHUNK + k) * OCH, OCH)]

    fire_tok(0, 0)
    fire_tok(1, 1)

    def process_chunk(k, par):
        pltpu.make_async_copy(
            tok_hbm.at[pl.ds((bs * NCHUNK + k) * TCH, TCH)], tb[par],
            tsem[par]).wait()

        @pl.when(k >= 2)
        def _():
            pltpu.make_async_copy(ob[par], out_slice(k - 2), osem[par]).wait()

        def rg_body(rg, _):
            zero = jnp.zeros((16,), jnp.float32)
            one = jnp.ones((16,), jnp.float32)
            accs = [zero] * C
            cnt = zero
            for j in range(L):
                tok = tb[par][pl.ds(j * CHUNK + rg * 16, 16)]
                tokc = tok * C
                cnt = cnt + jnp.where(tok != 0, one, zero)
                for c in range(C):
                    accs[c] = accs[c] + plsc.load_gather(ts, [tokc + c])
            scale = 1.0 / jnp.maximum(cnt, 1.0)
            base_idx = lanes * C + rg * (16 * C)
            for c in range(C):
                plsc.store_scatter(ob[par], [base_idx + c], accs[c] * scale)
            return 0

        lax.fori_loop(0, CHUNK // 16, rg_body, 0)
        pltpu.async_copy(ob[par], out_slice(k), osem[par])

        @pl.when(k + 2 < NCHUNK)
        def _():
            fire_tok(k + 2, par)

    def chunk_loop(g, _):
        for par in (0, 1):
            process_chunk(2 * g + par, par)
        return 0

    lax.fori_loop(0, NCHUNK // 2, chunk_loop, 0)
    for par in (0, 1):
        pltpu.make_async_copy(ob[par], out_slice(NCHUNK - 2 + par),
                              osem[par]).wait()


@jax.jit
def kernel(tokens, table):
    tab_r = table.reshape(VOCAB, NCG, C).transpose(1, 0, 2).reshape(-1)
    tok_r = tokens.reshape(NBS, NCHUNK, CHUNK, L).transpose(0, 1, 3, 2).reshape(-1)
    mesh = plsc.VectorSubcoreMesh(core_axis_name="c", subcore_axis_name="s")
    f = pl.kernel(
        _sc_body,
        out_type=jax.ShapeDtypeStruct((B * D,), jnp.float32),
        mesh=mesh,
        compiler_params=pltpu.CompilerParams(needs_layout_passes=False),
        scratch_types=[
            pltpu.VMEM((TSL,), jnp.float32),
            [pltpu.VMEM((TCH,), jnp.int32) for _ in range(2)],
            [pltpu.VMEM((OCH,), jnp.float32) for _ in range(2)],
            [pltpu.SemaphoreType.DMA for _ in range(2)],
            [pltpu.SemaphoreType.DMA for _ in range(2)],
        ],
    )
    out_r = f(tok_r, tab_r)
    return (out_r.reshape(NCG, NBS, NCHUNK, CHUNK, C)
            .transpose(1, 2, 3, 0, 4).reshape(B, D))
